# baseline (speedup 1.0000x reference)
import functools

import jax
import jax.numpy as jnp
from jax import lax
from jax.experimental import pallas as pl
from jax.experimental.pallas import tpu as pltpu
from jax.experimental.pallas import tpu_sc as plsc

VOCAB = 1_000_000
D = 128
B = 16_384
NEG = 20

NC, NS = 2, 16
NW = NC * NS
CHUNK = 128

BPW = B // NW
T_CH = BPW // CHUNK
C_CH = BPW // CHUNK
NG = 4
NROWS = NG * NEG
N_CH = BPW // NG


@functools.partial(
    pl.kernel,
    out_type=(
        jax.ShapeDtypeStruct((B, D), jnp.float32),
        jax.ShapeDtypeStruct((B, D), jnp.float32),
        jax.ShapeDtypeStruct((B, NEG, D), jnp.float32),
    ),
    mesh=plsc.VectorSubcoreMesh(core_axis_name="c", subcore_axis_name="s"),
    scratch_types=[
        pltpu.VMEM((T_CH + C_CH, CHUNK), jnp.int32),
        pltpu.VMEM((N_CH, NROWS), jnp.int32),
        pltpu.VMEM((CHUNK, D), jnp.float32),
        pltpu.VMEM((NROWS, D), jnp.float32),
        pltpu.SemaphoreType.DMA,
    ],
)
def _gather_all(t_emb, c_emb, idx_tc_hbm, idx_n_hbm, t_out, c_out, n_out,
                idx_tc, idx_n, rows_v, nrows_v, sem):
    wid = lax.axis_index("s") * NC + lax.axis_index("c")
    pltpu.sync_copy(idx_tc_hbm.at[wid], idx_tc)
    pltpu.sync_copy(idx_n_hbm.at[wid], idx_n)

    t_base = wid * (T_CH * CHUNK)
    for j in range(T_CH):
        pltpu.async_copy(t_emb.at[idx_tc.at[j]], rows_v, sem).wait()
        pltpu.sync_copy(rows_v, t_out.at[pl.ds(t_base + j * CHUNK, CHUNK)])

    c_base = wid * (C_CH * CHUNK)
    for j in range(C_CH):
        pltpu.async_copy(c_emb.at[idx_tc.at[T_CH + j]], rows_v, sem).wait()
        pltpu.sync_copy(rows_v, c_out.at[pl.ds(c_base + j * CHUNK, CHUNK)])

    n_base = wid * BPW

    @pl.loop(0, N_CH)
    def _neg(j):
        pltpu.async_copy(c_emb.at[idx_n.at[j]], nrows_v, sem).wait()
        for g in range(NG):
            pltpu.sync_copy(nrows_v.at[pl.ds(g * NEG, NEG)],
                            n_out.at[n_base + j * NG + g])


def kernel(target_ids, context_ids, negative_ids, target_emb, context_emb):
    tid = target_ids.astype(jnp.int32).reshape(NW, T_CH, CHUNK)
    cid = context_ids.astype(jnp.int32).reshape(NW, C_CH, CHUNK)
    idx_tc = jnp.concatenate([tid, cid], axis=1)
    idx_n = negative_ids.astype(jnp.int32).reshape(NW, N_CH, NROWS)
    target_vector, context_vector, negative_vector = _gather_all(
        target_emb, context_emb, idx_tc, idx_n)
    return (target_vector, context_vector, negative_vector)


# speedup vs baseline: 1.4233x; 1.4233x over previous
import functools

import jax
import jax.numpy as jnp
from jax import lax
from jax.experimental import pallas as pl
from jax.experimental.pallas import tpu as pltpu
from jax.experimental.pallas import tpu_sc as plsc

VOCAB = 1_000_000
D = 128
B = 16_384
NEG = 20

NC, NS = 2, 16
NW = NC * NS
BPW = B // NW

TCK = 64
TC_CH = 2 * BPW // TCK

NG = 4
NROWS = NG * NEG
N_CH = BPW // NG

NBUF = 4


@functools.partial(
    pl.kernel,
    out_type=(
        jax.ShapeDtypeStruct((B, D), jnp.float32),
        jax.ShapeDtypeStruct((B, D), jnp.float32),
        jax.ShapeDtypeStruct((B, NEG, D), jnp.float32),
    ),
    mesh=plsc.VectorSubcoreMesh(core_axis_name="c", subcore_axis_name="s"),
    scratch_types=(
        [
            pltpu.VMEM((TC_CH, TCK), jnp.int32),
            pltpu.VMEM((N_CH, NROWS), jnp.int32),
        ]
        + [pltpu.VMEM((NROWS, D), jnp.float32)] * NBUF
        + [pltpu.SemaphoreType.DMA] * (2 * NBUF)
    ),
)
def _gather_all(t_emb, c_emb, idx_tc_hbm, idx_n_hbm, t_out, c_out, n_out,
                idx_tc, idx_n, *bufs_and_sems):
    bufs = bufs_and_sems[:NBUF]
    sg = bufs_and_sems[NBUF:2 * NBUF]
    sw = bufs_and_sems[2 * NBUF:3 * NBUF]

    wid = lax.axis_index("s") * NC + lax.axis_index("c")
    pltpu.sync_copy(idx_tc_hbm.at[wid], idx_tc)
    pltpu.sync_copy(idx_n_hbm.at[wid], idx_n)

    base = wid * BPW
    for j in range(0, TC_CH, NBUF):
        descs = []
        for b in range(NBUF):
            k = j + b
            if j > 0:
                pltpu.make_async_copy(
                    c_emb.at[pl.ds(0, TCK)], bufs[b].at[pl.ds(0, TCK)], sw[b]
                ).wait()
            table = t_emb if k < TC_CH // 2 else c_emb
            descs.append(pltpu.async_copy(
                table.at[idx_tc.at[k]], bufs[b].at[pl.ds(0, TCK)], sg[b]))
        for b in range(NBUF):
            k = j + b
            descs[b].wait()
            out = t_out if k < TC_CH // 2 else c_out
            off = base + (k % (TC_CH // 2)) * TCK
            pltpu.async_copy(bufs[b].at[pl.ds(0, TCK)],
                             out.at[pl.ds(off, TCK)], sw[b])
    for b in range(NBUF):
        pltpu.make_async_copy(
            c_emb.at[pl.ds(0, TCK)], bufs[b].at[pl.ds(0, TCK)], sw[b]
        ).wait()

    n_base = wid * BPW

    @pl.loop(0, N_CH, step=NBUF)
    def _grp(j):
        descs = []
        for b in range(NBUF):
            @pl.when(j > 0)
            def _drain():
                pltpu.make_async_copy(
                    c_emb.at[pl.ds(0, NROWS)], bufs[b], sw[b]).wait()
            descs.append(
                pltpu.async_copy(c_emb.at[idx_n.at[j + b]], bufs[b], sg[b]))
        for b in range(NBUF):
            descs[b].wait()
            for g in range(NG):
                pltpu.async_copy(bufs[b].at[pl.ds(g * NEG, NEG)],
                                 n_out.at[n_base + (j + b) * NG + g], sw[b])

    for b in range(NBUF):
        pltpu.make_async_copy(
            c_emb.at[pl.ds(0, NROWS)], bufs[b], sw[b]).wait()


def kernel(target_ids, context_ids, negative_ids, target_emb, context_emb):
    tid = target_ids.astype(jnp.int32).reshape(NW, TC_CH // 2, TCK)
    cid = context_ids.astype(jnp.int32).reshape(NW, TC_CH // 2, TCK)
    idx_tc = jnp.concatenate([tid, cid], axis=1)
    idx_n = negative_ids.astype(jnp.int32).reshape(NW, N_CH, NROWS)
    target_vector, context_vector, negative_vector = _gather_all(
        target_emb, context_emb, idx_tc, idx_n)
    return (target_vector, context_vector, negative_vector)


# speedup vs baseline: 2.6068x; 1.8315x over previous
import functools

import jax
import jax.numpy as jnp
from jax import lax
from jax.experimental import pallas as pl
from jax.experimental.pallas import tpu as pltpu
from jax.experimental.pallas import tpu_sc as plsc

VOCAB = 1_000_000
D = 128
B = 16_384
NEG = 20

NC, NS = 2, 16
NW = NC * NS
BPW = B // NW
CHUNK = 128
CPS = BPW // CHUNK
N_CH = NEG * CPS
ALL_CH = 2 * CPS + N_CH

NBUF = 4


@functools.partial(
    pl.kernel,
    out_type=(
        jax.ShapeDtypeStruct((B, D), jnp.float32),
        jax.ShapeDtypeStruct((B, D), jnp.float32),
        jax.ShapeDtypeStruct((NEG, B, D), jnp.float32),
    ),
    mesh=plsc.VectorSubcoreMesh(core_axis_name="c", subcore_axis_name="s"),
    scratch_types=(
        [pltpu.VMEM((ALL_CH, CHUNK), jnp.int32)]
        + [pltpu.VMEM((CHUNK, D), jnp.float32)] * NBUF
        + [pltpu.SemaphoreType.DMA] * (2 * NBUF)
    ),
)
def _gather_all(t_emb, c_emb, idx_hbm, t_out, c_out, n_out,
                idx_v, *bufs_and_sems):
    bufs = bufs_and_sems[:NBUF]
    sg = bufs_and_sems[NBUF:2 * NBUF]
    sw = bufs_and_sems[2 * NBUF:3 * NBUF]

    wid = lax.axis_index("s") * NC + lax.axis_index("c")
    pltpu.sync_copy(idx_hbm.at[wid], idx_v)
    base = wid * BPW

    for seg, (table, out) in enumerate(((t_emb, t_out), (c_emb, c_out))):
        descs = []
        for b in range(NBUF):
            if seg > 0:
                pltpu.make_async_copy(c_emb.at[pl.ds(0, CHUNK)],
                                      bufs[b], sw[b]).wait()
            descs.append(pltpu.async_copy(
                table.at[idx_v.at[seg * CPS + b]], bufs[b], sg[b]))
        for b in range(NBUF):
            descs[b].wait()
            pltpu.async_copy(bufs[b],
                             out.at[pl.ds(base + b * CHUNK, CHUNK)], sw[b])

    @pl.loop(0, N_CH, step=NBUF)
    def _grp(j):
        g = j // CPS
        descs = []
        for b in range(NBUF):
            pltpu.make_async_copy(c_emb.at[pl.ds(0, CHUNK)],
                                  bufs[b], sw[b]).wait()
            descs.append(pltpu.async_copy(
                c_emb.at[idx_v.at[2 * CPS + j + b]], bufs[b], sg[b]))
        for b in range(NBUF):
            descs[b].wait()
            pltpu.async_copy(
                bufs[b], n_out.at[g, pl.ds(base + b * CHUNK, CHUNK)], sw[b])

    for b in range(NBUF):
        pltpu.make_async_copy(c_emb.at[pl.ds(0, CHUNK)], bufs[b], sw[b]).wait()


def kernel(target_ids, context_ids, negative_ids, target_emb, context_emb):
    tid = target_ids.astype(jnp.int32).reshape(NW, CPS, CHUNK)
    cid = context_ids.astype(jnp.int32).reshape(NW, CPS, CHUNK)
    nid = (negative_ids.astype(jnp.int32).T
           .reshape(NEG, NW, CPS, CHUNK)
           .transpose(1, 0, 2, 3)
           .reshape(NW, N_CH, CHUNK))
    idx = jnp.concatenate([tid, cid, nid], axis=1)
    target_vector, context_vector, neg_t = _gather_all(
        target_emb, context_emb, idx)
    return (target_vector, context_vector, neg_t.transpose(1, 0, 2))


# speedup vs baseline: 2.6090x; 1.0009x over previous
import functools

import jax
import jax.numpy as jnp
from jax import lax
from jax.experimental import pallas as pl
from jax.experimental.pallas import tpu as pltpu
from jax.experimental.pallas import tpu_sc as plsc

VOCAB = 1_000_000
D = 128
B = 16_384
NEG = 20

NC, NS = 2, 16
NW = NC * NS
BPW = B // NW
CHUNK = 128
CPS = BPW // CHUNK
N_CH = NEG * CPS
ALL_CH = 2 * CPS + N_CH

NBUF = 4


@functools.partial(
    pl.kernel,
    out_type=(
        jax.ShapeDtypeStruct((B, D), jnp.float32),
        jax.ShapeDtypeStruct((B, D), jnp.float32),
        jax.ShapeDtypeStruct((NEG, B, D), jnp.float32),
    ),
    mesh=plsc.VectorSubcoreMesh(core_axis_name="c", subcore_axis_name="s"),
    scratch_types=(
        [
            pltpu.VMEM((2 * CPS, CHUNK), jnp.int32),
            pltpu.VMEM((NEG, BPW), jnp.int32),
        ]
        + [pltpu.VMEM((CHUNK, D), jnp.float32)] * NBUF
        + [pltpu.SemaphoreType.DMA] * (2 * NBUF)
    ),
)
def _gather_all(t_emb, c_emb, tid_hbm, cid_hbm, nid_hbm, t_out, c_out, n_out,
                idx_tc, idx_n, *bufs_and_sems):
    bufs = bufs_and_sems[:NBUF]
    sg = bufs_and_sems[NBUF:2 * NBUF]
    sw = bufs_and_sems[2 * NBUF:3 * NBUF]

    wid = lax.axis_index("s") * NC + lax.axis_index("c")
    base = wid * BPW
    pltpu.sync_copy(tid_hbm.at[wid], idx_tc.at[pl.ds(0, CPS)])
    pltpu.sync_copy(cid_hbm.at[wid], idx_tc.at[pl.ds(CPS, CPS)])
    pltpu.sync_copy(nid_hbm.at[:, pl.ds(base, BPW)], idx_n)

    for seg, (table, out) in enumerate(((t_emb, t_out), (c_emb, c_out))):
        descs = []
        for b in range(NBUF):
            if seg > 0:
                pltpu.make_async_copy(c_emb.at[pl.ds(0, CHUNK)],
                                      bufs[b], sw[b]).wait()
            descs.append(pltpu.async_copy(
                table.at[idx_tc.at[seg * CPS + b]], bufs[b], sg[b]))
        for b in range(NBUF):
            descs[b].wait()
            pltpu.async_copy(bufs[b],
                             out.at[pl.ds(base + b * CHUNK, CHUNK)], sw[b])

    @pl.loop(0, N_CH, step=NBUF)
    def _grp(j):
        g = j // CPS
        descs = []
        for b in range(NBUF):
            pltpu.make_async_copy(c_emb.at[pl.ds(0, CHUNK)],
                                  bufs[b], sw[b]).wait()
            descs.append(pltpu.async_copy(
                c_emb.at[idx_n.at[g, pl.ds(b * CHUNK, CHUNK)]],
                bufs[b], sg[b]))
        for b in range(NBUF):
            descs[b].wait()
            pltpu.async_copy(
                bufs[b], n_out.at[g, pl.ds(base + b * CHUNK, CHUNK)], sw[b])

    for b in range(NBUF):
        pltpu.make_async_copy(c_emb.at[pl.ds(0, CHUNK)], bufs[b], sw[b]).wait()


def kernel(target_ids, context_ids, negative_ids, target_emb, context_emb):
    tid = target_ids.astype(jnp.int32).reshape(NW, CPS, CHUNK)
    cid = context_ids.astype(jnp.int32).reshape(NW, CPS, CHUNK)
    nid_t = negative_ids.astype(jnp.int32).T
    target_vector, context_vector, neg_t = _gather_all(
        target_emb, context_emb, tid, cid, nid_t)
    return (target_vector, context_vector, neg_t.transpose(1, 0, 2))


# speedup vs baseline: 2.6635x; 1.0209x over previous
import functools

import jax
import jax.numpy as jnp
from jax import lax
from jax.experimental import pallas as pl
from jax.experimental.pallas import tpu as pltpu
from jax.experimental.pallas import tpu_sc as plsc

VOCAB = 1_000_000
D = 128
B = 16_384
NEG = 20

NC, NS = 2, 16
NW = NC * NS
BPW = B // NW
CHUNK = 64
CPS = BPW // CHUNK
N_CH = NEG * CPS
ALL_CH = 2 * CPS + N_CH

NBUF = 8


@functools.partial(
    pl.kernel,
    out_type=(
        jax.ShapeDtypeStruct((B, D), jnp.float32),
        jax.ShapeDtypeStruct((B, D), jnp.float32),
        jax.ShapeDtypeStruct((NEG, B, D), jnp.float32),
    ),
    mesh=plsc.VectorSubcoreMesh(core_axis_name="c", subcore_axis_name="s"),
    scratch_types=(
        [
            pltpu.VMEM((2 * CPS, CHUNK), jnp.int32),
            pltpu.VMEM((NEG, BPW), jnp.int32),
        ]
        + [pltpu.VMEM((CHUNK, D), jnp.float32)] * NBUF
        + [pltpu.SemaphoreType.DMA] * (2 * NBUF)
    ),
)
def _gather_all(t_emb, c_emb, tid_hbm, cid_hbm, nid_hbm, t_out, c_out, n_out,
                idx_tc, idx_n, *bufs_and_sems):
    bufs = bufs_and_sems[:NBUF]
    sg = bufs_and_sems[NBUF:2 * NBUF]
    sw = bufs_and_sems[2 * NBUF:3 * NBUF]

    wid = lax.axis_index("s") * NC + lax.axis_index("c")
    base = wid * BPW
    pltpu.sync_copy(tid_hbm.at[wid], idx_tc.at[pl.ds(0, CPS)])
    pltpu.sync_copy(cid_hbm.at[wid], idx_tc.at[pl.ds(CPS, CPS)])
    pltpu.sync_copy(nid_hbm.at[:, pl.ds(base, BPW)], idx_n)

    for seg, (table, out) in enumerate(((t_emb, t_out), (c_emb, c_out))):
        descs = []
        for b in range(NBUF):
            if seg > 0:
                pltpu.make_async_copy(c_emb.at[pl.ds(0, CHUNK)],
                                      bufs[b], sw[b]).wait()
            descs.append(pltpu.async_copy(
                table.at[idx_tc.at[seg * CPS + b]], bufs[b], sg[b]))
        for b in range(NBUF):
            descs[b].wait()
            pltpu.async_copy(bufs[b],
                             out.at[pl.ds(base + b * CHUNK, CHUNK)], sw[b])

    @pl.loop(0, N_CH, step=NBUF)
    def _grp(j):
        g = j // CPS
        descs = []
        for b in range(NBUF):
            pltpu.make_async_copy(c_emb.at[pl.ds(0, CHUNK)],
                                  bufs[b], sw[b]).wait()
            descs.append(pltpu.async_copy(
                c_emb.at[idx_n.at[g, pl.ds(b * CHUNK, CHUNK)]],
                bufs[b], sg[b]))
        for b in range(NBUF):
            descs[b].wait()
            pltpu.async_copy(
                bufs[b], n_out.at[g, pl.ds(base + b * CHUNK, CHUNK)], sw[b])

    for b in range(NBUF):
        pltpu.make_async_copy(c_emb.at[pl.ds(0, CHUNK)], bufs[b], sw[b]).wait()


def kernel(target_ids, context_ids, negative_ids, target_emb, context_emb):
    tid = target_ids.astype(jnp.int32).reshape(NW, CPS, CHUNK)
    cid = context_ids.astype(jnp.int32).reshape(NW, CPS, CHUNK)
    nid_t = negative_ids.astype(jnp.int32).T
    target_vector, context_vector, neg_t = _gather_all(
        target_emb, context_emb, tid, cid, nid_t)
    return (target_vector, context_vector, neg_t.transpose(1, 0, 2))
